# baseline (speedup 1.0000x reference)
import jax
import jax.numpy as jnp
from jax.experimental import pallas as pl
from jax.experimental.pallas import tpu as pltpu

LANE = 128
SUBLANE = 8
TILE_B = 512
CHUNK_H = 1024
LAG = 2
NRING = LAG + 1


def _round_up(n, m):
    return (n + m - 1) // m * m


def _make_body(nb, n_chunks, chunk_h):
    def _mlp_body(x_ref, w1_hbm, b1_ref, w2_hbm, b2_ref, o_ref,
                  w1_v, w2_v, hbuf, sem1, sem2):
        i = pl.program_id(0)

        def _w1_copy(c):
            lo = c * chunk_h
            return pltpu.make_async_copy(
                w1_hbm.at[:, pl.ds(lo, chunk_h)],
                w1_v.at[:, pl.ds(lo, chunk_h)],
                sem1.at[c])

        def _w2_copy(c):
            lo = c * chunk_h
            return pltpu.make_async_copy(
                w2_hbm.at[pl.ds(lo, chunk_h), :],
                w2_v.at[pl.ds(lo, chunk_h), :],
                sem2.at[c])

        @pl.when(i == 0)
        def _start_loads():
            for c in range(n_chunks):
                _w1_copy(c).start()
            for c in range(n_chunks):
                _w2_copy(c).start()

        @pl.when(i < nb)
        def _mm1():
            x = x_ref[...]
            slot = jax.lax.rem(i, NRING)
            for c in range(n_chunks):
                @pl.when(i == 0)
                def _wait_w1(c=c):
                    _w1_copy(c).wait()
                lo = c * chunk_h
                hi = lo + chunk_h
                h = jnp.dot(x, w1_v[:, lo:hi],
                            preferred_element_type=jnp.float32)
                h = jnp.maximum(h + b1_ref[:, lo:hi], 0.0)
                hbuf[slot, :, lo:hi] = h.astype(jnp.bfloat16)

        @pl.when(i >= LAG)
        def _mm2():
            slot = jax.lax.rem(i - LAG, NRING)
            y = b2_ref[...]
            for c in range(n_chunks):
                @pl.when(i == LAG)
                def _wait_w2(c=c):
                    _w2_copy(c).wait()
                lo = c * chunk_h
                hi = lo + chunk_h
                y = y + jnp.dot(hbuf[slot, :, lo:hi], w2_v[lo:hi, :],
                                preferred_element_type=jnp.float32)
            o_ref[...] = y
    return _mlp_body


def _forward(x, w1_p, b1_p, w2_p, b2_p):
    B, d_in = x.shape
    d_in_p, h_p = w1_p.shape
    _, d_out_p = w2_p.shape

    tile_b = min(TILE_B, _round_up(B, SUBLANE))
    b_pad = _round_up(B, tile_b)
    nb = b_pad // tile_b
    if h_p % CHUNK_H == 0:
        chunk_h, n_chunks = CHUNK_H, h_p // CHUNK_H
    else:
        chunk_h, n_chunks = h_p, 1

    if (b_pad, d_in_p) == (B, d_in):
        x_p = x
    else:
        x_p = jnp.zeros((b_pad, d_in_p), x.dtype).at[:B, :d_in].set(x)

    flops = 2 * b_pad * (d_in_p * h_p + h_p * d_out_p)
    bytes_accessed = 4 * (
        b_pad * d_in_p
        + d_in_p * h_p + h_p
        + h_p * d_out_p + d_out_p
        + b_pad * d_out_p
    )

    out_p = pl.pallas_call(
        _make_body(nb, n_chunks, chunk_h),
        out_shape=jax.ShapeDtypeStruct((b_pad, d_out_p), jnp.float32),
        grid_spec=pltpu.PrefetchScalarGridSpec(
            num_scalar_prefetch=0,
            grid=(nb + LAG,),
            in_specs=[
                pl.BlockSpec((tile_b, d_in_p),
                             lambda i: (jnp.minimum(i, nb - 1), 0)),
                pl.BlockSpec(memory_space=pl.ANY),
                pl.BlockSpec((1, h_p), lambda i: (0, 0)),
                pl.BlockSpec(memory_space=pl.ANY),
                pl.BlockSpec((1, d_out_p), lambda i: (0, 0)),
            ],
            out_specs=pl.BlockSpec(
                (tile_b, d_out_p),
                lambda i: (jnp.clip(i - LAG, 0, nb - 1), 0)),
            scratch_shapes=[
                pltpu.MemorySpace.VMEM((d_in_p, h_p), jnp.float32),
                pltpu.MemorySpace.VMEM((h_p, d_out_p), jnp.float32),
                pltpu.MemorySpace.VMEM((NRING, tile_b, h_p), jnp.bfloat16),
                pltpu.SemaphoreType.DMA((n_chunks,)),
                pltpu.SemaphoreType.DMA((n_chunks,)),
            ],
        ),
        compiler_params=pltpu.CompilerParams(
            dimension_semantics=("arbitrary",),
        ),
        cost_estimate=pl.CostEstimate(
            flops=flops, transcendentals=0, bytes_accessed=bytes_accessed
        ),
    )(x_p, w1_p, b1_p, w2_p, b2_p)

    return out_p[:B, :]


def kernel(x, w1_p, b1_p, w2_p, b2_p):
    d_out = 1024
    return _forward(x, w1_p, b1_p, w2_p, b2_p)[:, :d_out]


# speedup vs baseline: 1.0993x; 1.0993x over previous
import jax
import jax.numpy as jnp
from jax.experimental import pallas as pl
from jax.experimental.pallas import tpu as pltpu

LANE = 128
SUBLANE = 8
TILE_B = 512
CHUNK_H = 1024


def _round_up(n, m):
    return (n + m - 1) // m * m


def _make_body(n_chunks, chunk_h):
    def _mlp_body(x_ref, w1_ref, b1_ref, w2_hbm, b2_ref, o_ref,
                  w2_v, sem2):
        i = pl.program_id(0)

        def _w2_copy(c):
            lo = c * chunk_h
            return pltpu.make_async_copy(
                w2_hbm.at[pl.ds(lo, chunk_h), :],
                w2_v.at[pl.ds(lo, chunk_h), :],
                sem2.at[c])

        @pl.when(i == 0)
        def _start_loads():
            for c in range(n_chunks):
                _w2_copy(c).start()

        h = jnp.dot(x_ref[...], w1_ref[...],
                    preferred_element_type=jnp.float32)
        h = jnp.maximum(h + b1_ref[...], 0.0)
        y = b2_ref[...]
        for c in range(n_chunks):
            @pl.when(i == 0)
            def _wait_w2(c=c):
                _w2_copy(c).wait()
            lo = c * chunk_h
            hi = lo + chunk_h
            y = y + jnp.dot(h[:, lo:hi], w2_v[lo:hi, :],
                            preferred_element_type=jnp.float32)
        o_ref[...] = y
    return _mlp_body


def _forward(x, w1_p, b1_p, w2_p, b2_p):
    B, d_in = x.shape
    d_in_p, h_p = w1_p.shape
    _, d_out_p = w2_p.shape

    tile_b = min(TILE_B, _round_up(B, SUBLANE))
    b_pad = _round_up(B, tile_b)
    nb = b_pad // tile_b
    if h_p % CHUNK_H == 0:
        chunk_h, n_chunks = CHUNK_H, h_p // CHUNK_H
    else:
        chunk_h, n_chunks = h_p, 1

    if (b_pad, d_in_p) == (B, d_in):
        x_p = x
    else:
        x_p = jnp.zeros((b_pad, d_in_p), x.dtype).at[:B, :d_in].set(x)

    flops = 2 * b_pad * (d_in_p * h_p + h_p * d_out_p)
    bytes_accessed = 4 * (
        b_pad * d_in_p
        + d_in_p * h_p + h_p
        + h_p * d_out_p + d_out_p
        + b_pad * d_out_p
    )

    out_p = pl.pallas_call(
        _make_body(n_chunks, chunk_h),
        out_shape=jax.ShapeDtypeStruct((b_pad, d_out_p), jnp.float32),
        grid_spec=pltpu.PrefetchScalarGridSpec(
            num_scalar_prefetch=0,
            grid=(nb,),
            in_specs=[
                pl.BlockSpec((tile_b, d_in_p), lambda i: (i, 0)),
                pl.BlockSpec((d_in_p, h_p), lambda i: (0, 0)),
                pl.BlockSpec((1, h_p), lambda i: (0, 0)),
                pl.BlockSpec(memory_space=pl.ANY),
                pl.BlockSpec((1, d_out_p), lambda i: (0, 0)),
            ],
            out_specs=pl.BlockSpec((tile_b, d_out_p), lambda i: (i, 0)),
            scratch_shapes=[
                pltpu.MemorySpace.VMEM((h_p, d_out_p), jnp.float32),
                pltpu.SemaphoreType.DMA((n_chunks,)),
            ],
        ),
        compiler_params=pltpu.CompilerParams(
            dimension_semantics=("arbitrary",),
        ),
        cost_estimate=pl.CostEstimate(
            flops=flops, transcendentals=0, bytes_accessed=bytes_accessed
        ),
    )(x_p, w1_p, b1_p, w2_p, b2_p)

    return out_p[:B, :]


def kernel(x, w1_p, b1_p, w2_p, b2_p):
    d_out = 1024
    return _forward(x, w1_p, b1_p, w2_p, b2_p)[:, :d_out]


# speedup vs baseline: 1.1087x; 1.0086x over previous
import jax
import jax.numpy as jnp
from jax.experimental import pallas as pl
from jax.experimental.pallas import tpu as pltpu

LANE = 128
SUBLANE = 8
TILE_B = 512
CHUNK_H = 1024


def _round_up(n, m):
    return (n + m - 1) // m * m


def _make_body(n_chunks, chunk_h):
    def _mlp_body(x_ref, w1_ref, b1_ref, w2_ref, b2_ref, o_ref):
        x = x_ref[...]
        y = b2_ref[...]
        for c in range(n_chunks):
            lo = c * chunk_h
            hi = lo + chunk_h
            h = jnp.dot(x, w1_ref[:, lo:hi],
                        preferred_element_type=jnp.float32)
            h = jnp.maximum(h + b1_ref[:, lo:hi], 0.0)
            y = y + jnp.dot(h, w2_ref[lo:hi, :],
                            preferred_element_type=jnp.float32)
        o_ref[...] = y
    return _mlp_body


def _forward(x, w1_p, b1_p, w2_p, b2_p):
    B, d_in = x.shape
    d_in_p, h_p = w1_p.shape
    _, d_out_p = w2_p.shape

    tile_b = min(TILE_B, _round_up(B, SUBLANE))
    b_pad = _round_up(B, tile_b)
    nb = b_pad // tile_b
    if h_p % CHUNK_H == 0:
        chunk_h, n_chunks = CHUNK_H, h_p // CHUNK_H
    else:
        chunk_h, n_chunks = h_p, 1

    if (b_pad, d_in_p) == (B, d_in):
        x_p = x
    else:
        x_p = jnp.zeros((b_pad, d_in_p), x.dtype).at[:B, :d_in].set(x)

    flops = 2 * b_pad * (d_in_p * h_p + h_p * d_out_p)
    bytes_accessed = 4 * (
        b_pad * d_in_p
        + d_in_p * h_p + h_p
        + h_p * d_out_p + d_out_p
        + b_pad * d_out_p
    )

    out_p = pl.pallas_call(
        _make_body(n_chunks, chunk_h),
        out_shape=jax.ShapeDtypeStruct((b_pad, d_out_p), jnp.float32),
        grid_spec=pltpu.PrefetchScalarGridSpec(
            num_scalar_prefetch=0,
            grid=(nb,),
            in_specs=[
                pl.BlockSpec((tile_b, d_in_p), lambda i: (i, 0)),
                pl.BlockSpec((d_in_p, h_p), lambda i: (0, 0)),
                pl.BlockSpec((1, h_p), lambda i: (0, 0)),
                pl.BlockSpec((h_p, d_out_p), lambda i: (0, 0)),
                pl.BlockSpec((1, d_out_p), lambda i: (0, 0)),
            ],
            out_specs=pl.BlockSpec((tile_b, d_out_p), lambda i: (i, 0)),
        ),
        compiler_params=pltpu.CompilerParams(
            dimension_semantics=("parallel",),
        ),
        cost_estimate=pl.CostEstimate(
            flops=flops, transcendentals=0, bytes_accessed=bytes_accessed
        ),
    )(x_p, w1_p, b1_p, w2_p, b2_p)

    return out_p[:B, :]


def kernel(x, w1_p, b1_p, w2_p, b2_p):
    d_out = 1024
    return _forward(x, w1_p, b1_p, w2_p, b2_p)[:, :d_out]


# speedup vs baseline: 1.1170x; 1.0075x over previous
import jax
import jax.numpy as jnp
from jax.experimental import pallas as pl
from jax.experimental.pallas import tpu as pltpu

LANE = 128
SUBLANE = 8
TILE_B = 1024
CHUNK_H = 1024


def _round_up(n, m):
    return (n + m - 1) // m * m


def _make_body(n_chunks, chunk_h):
    def _mlp_body(x_ref, w1_ref, b1_ref, w2_ref, b2_ref, o_ref):
        x = x_ref[...]
        y = b2_ref[...]
        for c in range(n_chunks):
            lo = c * chunk_h
            hi = lo + chunk_h
            h = jnp.dot(x, w1_ref[:, lo:hi],
                        preferred_element_type=jnp.float32)
            h = jnp.maximum(h + b1_ref[:, lo:hi], 0.0)
            y = y + jnp.dot(h, w2_ref[lo:hi, :],
                            preferred_element_type=jnp.float32)
        o_ref[...] = y
    return _mlp_body


def _forward(x, w1_p, b1_p, w2_p, b2_p):
    B, d_in = x.shape
    d_in_p, h_p = w1_p.shape
    _, d_out_p = w2_p.shape

    tile_b = min(TILE_B, _round_up(B, SUBLANE))
    b_pad = _round_up(B, tile_b)
    nb = b_pad // tile_b
    if h_p % CHUNK_H == 0:
        chunk_h, n_chunks = CHUNK_H, h_p // CHUNK_H
    else:
        chunk_h, n_chunks = h_p, 1

    if (b_pad, d_in_p) == (B, d_in):
        x_p = x
    else:
        x_p = jnp.zeros((b_pad, d_in_p), x.dtype).at[:B, :d_in].set(x)

    flops = 2 * b_pad * (d_in_p * h_p + h_p * d_out_p)
    bytes_accessed = 4 * (
        b_pad * d_in_p
        + d_in_p * h_p + h_p
        + h_p * d_out_p + d_out_p
        + b_pad * d_out_p
    )

    out_p = pl.pallas_call(
        _make_body(n_chunks, chunk_h),
        out_shape=jax.ShapeDtypeStruct((b_pad, d_out_p), jnp.float32),
        grid_spec=pltpu.PrefetchScalarGridSpec(
            num_scalar_prefetch=0,
            grid=(nb,),
            in_specs=[
                pl.BlockSpec((tile_b, d_in_p), lambda i: (i, 0)),
                pl.BlockSpec((d_in_p, h_p), lambda i: (0, 0)),
                pl.BlockSpec((1, h_p), lambda i: (0, 0)),
                pl.BlockSpec((h_p, d_out_p), lambda i: (0, 0)),
                pl.BlockSpec((1, d_out_p), lambda i: (0, 0)),
            ],
            out_specs=pl.BlockSpec((tile_b, d_out_p), lambda i: (i, 0)),
        ),
        compiler_params=pltpu.CompilerParams(
            dimension_semantics=("parallel",),
        ),
        cost_estimate=pl.CostEstimate(
            flops=flops, transcendentals=0, bytes_accessed=bytes_accessed
        ),
    )(x_p, w1_p, b1_p, w2_p, b2_p)

    return out_p[:B, :]


def kernel(x, w1_p, b1_p, w2_p, b2_p):
    d_out = 1024
    return _forward(x, w1_p, b1_p, w2_p, b2_p)[:, :d_out]


# speedup vs baseline: 1.1240x; 1.0062x over previous
import jax
import jax.numpy as jnp
from jax.experimental import pallas as pl
from jax.experimental.pallas import tpu as pltpu

LANE = 128
SUBLANE = 8
TILE_B = 1024
CHUNK_H = 1024


def _round_up(n, m):
    return (n + m - 1) // m * m


def _make_body(n_chunks, chunk_h):
    def _mlp_body(x_ref, w1_ref, b1_ref, w2_ref, b2_ref, o_ref):
        x = x_ref[...]
        y = b2_ref[...]
        for c in range(n_chunks):
            lo = c * chunk_h
            hi = lo + chunk_h
            h = jnp.dot(x, w1_ref[:, lo:hi],
                        preferred_element_type=jnp.float32)
            h = jnp.maximum(h + b1_ref[:, lo:hi], 0.0).astype(jnp.bfloat16)
            y = y + jnp.dot(h, w2_ref[lo:hi, :].astype(jnp.bfloat16),
                            preferred_element_type=jnp.float32)
        o_ref[...] = y
    return _mlp_body


def _forward(x, w1_p, b1_p, w2_p, b2_p):
    B, d_in = x.shape
    d_in_p, h_p = w1_p.shape
    _, d_out_p = w2_p.shape

    tile_b = min(TILE_B, _round_up(B, SUBLANE))
    b_pad = _round_up(B, tile_b)
    nb = b_pad // tile_b
    if h_p % CHUNK_H == 0:
        chunk_h, n_chunks = CHUNK_H, h_p // CHUNK_H
    else:
        chunk_h, n_chunks = h_p, 1

    if (b_pad, d_in_p) == (B, d_in):
        x_p = x
    else:
        x_p = jnp.zeros((b_pad, d_in_p), x.dtype).at[:B, :d_in].set(x)

    flops = 2 * b_pad * (d_in_p * h_p + h_p * d_out_p)
    bytes_accessed = 4 * (
        b_pad * d_in_p
        + d_in_p * h_p + h_p
        + h_p * d_out_p + d_out_p
        + b_pad * d_out_p
    )

    out_p = pl.pallas_call(
        _make_body(n_chunks, chunk_h),
        out_shape=jax.ShapeDtypeStruct((b_pad, d_out_p), jnp.float32),
        grid_spec=pltpu.PrefetchScalarGridSpec(
            num_scalar_prefetch=0,
            grid=(nb,),
            in_specs=[
                pl.BlockSpec((tile_b, d_in_p), lambda i: (i, 0)),
                pl.BlockSpec((d_in_p, h_p), lambda i: (0, 0)),
                pl.BlockSpec((1, h_p), lambda i: (0, 0)),
                pl.BlockSpec((h_p, d_out_p), lambda i: (0, 0)),
                pl.BlockSpec((1, d_out_p), lambda i: (0, 0)),
            ],
            out_specs=pl.BlockSpec((tile_b, d_out_p), lambda i: (i, 0)),
        ),
        compiler_params=pltpu.CompilerParams(
            dimension_semantics=("parallel",),
        ),
        cost_estimate=pl.CostEstimate(
            flops=flops, transcendentals=0, bytes_accessed=bytes_accessed
        ),
    )(x_p, w1_p, b1_p, w2_p, b2_p)

    return out_p[:B, :]


def kernel(x, w1_p, b1_p, w2_p, b2_p):
    d_out = 1024
    return _forward(x, w1_p, b1_p, w2_p, b2_p)[:, :d_out]
